# baseline (speedup 1.0000x reference)
import functools

import jax
import jax.numpy as jnp
from jax import lax
from jax.experimental import pallas as pl
from jax.experimental.pallas import tpu as pltpu
from jax.experimental.pallas import tpu_sc as plsc

_MARGIN = 0.1
_N = 4096
_L = 16
_NC = 2
_NS = 16
_NW = _NC * _NS
_RPW = _N // _NW
_CHUNK = 8
_NCH = _RPW // _CHUNK
_SLICES = _N // _L


def _sc_body(cos_hbm, mask_hbm, out_hbm, cos_buf, mask_buf, res_buf):
    cid = lax.axis_index("c")
    sid = lax.axis_index("s")
    wid = sid * _NC + cid
    base = wid * _RPW
    lane = lax.iota(jnp.int32, _L)
    zero = jnp.zeros((_L,), jnp.float32)

    def chunk_body(k, acc):
        r0 = base + k * _CHUNK
        pltpu.sync_copy(cos_hbm.at[pl.ds(r0, _CHUNK)], cos_buf)
        pltpu.sync_copy(mask_hbm.at[pl.ds(r0, _CHUNK)], mask_buf)
        for r in range(_CHUNK):
            def islice(j, c4):
                s_m, s_cm, s_r, s_rm = c4
                sl = pl.ds(j * _L, _L)
                c = cos_buf[r, sl]
                mf = mask_buf[r, sl].astype(jnp.float32)
                s_m = s_m + mf
                s_cm = s_cm + c * mf
                rr = jnp.maximum(c - _MARGIN, 0.0)
                s_r = s_r + rr
                s_rm = s_rm + rr * mf
                return (s_m, s_cm, s_r, s_rm)

            s_m, s_cm, s_r, s_rm = lax.fori_loop(
                0, _SLICES, islice, (zero, zero, zero, zero))
            cnt = jnp.sum(s_m)
            scm = jnp.sum(s_cm)
            sr = jnp.sum(s_r)
            srm = jnp.sum(s_rm)
            is0 = lane == 0
            num = jnp.where(is0, cnt - scm, sr - srm)
            den = jnp.where(is0, cnt, jnp.float32(_N) - cnt)
            frac = num / den
            keep = lane < 2
            acc = acc + jnp.where(keep, frac, 0.0)
        return acc

    acc = lax.fori_loop(0, _NCH, chunk_body, zero)
    res_buf[...] = acc
    pltpu.sync_copy(res_buf, out_hbm.at[wid])


def _sc_partials(cos_pred, mask_gt):
    mesh = plsc.VectorSubcoreMesh(
        core_axis_name="c", subcore_axis_name="s",
        num_cores=_NC, num_subcores=_NS)
    run = pl.kernel(
        _sc_body,
        out_type=jax.ShapeDtypeStruct((_NW, _L), jnp.float32),
        mesh=mesh,
        scratch_types=[
            pltpu.VMEM((_CHUNK, _N), jnp.float32),
            pltpu.VMEM((_CHUNK, _N), jnp.int32),
            pltpu.VMEM((_L,), jnp.float32),
        ],
        compiler_params=pltpu.CompilerParams(needs_layout_passes=False),
    )
    return run(cos_pred, mask_gt.astype(jnp.int32))


def kernel(cos_pred, mask_gt):
    parts = _sc_partials(cos_pred, mask_gt)
    lt_mean = jnp.sum(parts[:, 0]) / _N
    lnt_mean = jnp.sum(parts[:, 1]) / _N
    return ((lt_mean + lnt_mean) * 0.5, lt_mean, lnt_mean)


# speedup vs baseline: 2.8455x; 2.8455x over previous
import jax
import jax.numpy as jnp
from jax import lax
from jax.experimental import pallas as pl
from jax.experimental.pallas import tpu as pltpu
from jax.experimental.pallas import tpu_sc as plsc

_MARGIN = 0.1
_N = 4096

_R_SC = 768
_R_TC = _N - _R_SC

_L = 16
_NC = 2
_NS = 16
_NW = _NC * _NS
_RPW = _R_SC // _NW
_CHUNK = 4
_NCH = _RPW // _CHUNK
_SLICES = _N // _L
_UNROLL = 4

_BM = 256
_NBLK_TC = _R_TC // _BM

assert _R_SC % (_NW * _CHUNK) == 0 and _NCH % 2 == 0
assert _R_TC % _BM == 0


def _sc_body(cos_hbm, mask_hbm, out_hbm, cos_buf, mask_buf, res_buf,
             sem0, sem1):
    cid = lax.axis_index("c")
    sid = lax.axis_index("s")
    wid = sid * _NC + cid
    base = _R_TC + wid * _RPW
    lane = lax.iota(jnp.int32, _L)
    zero = jnp.zeros((_L,), jnp.float32)
    sems = (sem0, sem1)

    def start_chunk(k, b):
        r0 = base + k * _CHUNK
        pltpu.make_async_copy(
            cos_hbm.at[pl.ds(r0, _CHUNK)], cos_buf.at[b], sems[b]).start()
        pltpu.make_async_copy(
            mask_hbm.at[pl.ds(r0, _CHUNK)], mask_buf.at[b], sems[b]).start()

    def wait_chunk(b):
        pltpu.make_async_copy(
            cos_hbm.at[pl.ds(base, _CHUNK)], cos_buf.at[b], sems[b]).wait()
        pltpu.make_async_copy(
            mask_hbm.at[pl.ds(base, _CHUNK)], mask_buf.at[b], sems[b]).wait()

    def compute_chunk(b, acc):
        for r in range(_CHUNK):
            def islice(j, c4):
                s_m, s_cm, s_r, s_rm = c4
                for u in range(_UNROLL):
                    sl = pl.ds((j * _UNROLL + u) * _L, _L)
                    c = cos_buf[b, r, sl]
                    mf = mask_buf[b, r, sl].astype(jnp.float32)
                    s_m = s_m + mf
                    s_cm = s_cm + c * mf
                    rr = jnp.maximum(c - _MARGIN, 0.0)
                    s_r = s_r + rr
                    s_rm = s_rm + rr * mf
                return (s_m, s_cm, s_r, s_rm)

            s_m, s_cm, s_r, s_rm = lax.fori_loop(
                0, _SLICES // _UNROLL, islice, (zero, zero, zero, zero))
            cnt = jnp.sum(s_m)
            scm = jnp.sum(s_cm)
            sr = jnp.sum(s_r)
            srm = jnp.sum(s_rm)
            is0 = lane == 0
            num = jnp.where(is0, cnt - scm, sr - srm)
            den = jnp.where(is0, cnt, jnp.float32(_N) - cnt)
            acc = acc + jnp.where(lane < 2, num / den, 0.0)
        return acc

    start_chunk(0, 0)
    start_chunk(1, 1)

    def pair_body(j, acc):
        for b in (0, 1):
            k = j * 2 + b
            wait_chunk(b)
            acc = compute_chunk(b, acc)

            @pl.when(k + 2 < _NCH)
            def _():
                start_chunk(k + 2, b)
        return acc

    acc = lax.fori_loop(0, _NCH // 2, pair_body, zero)
    res_buf[...] = acc
    pltpu.sync_copy(res_buf, out_hbm.at[wid])


def _sc_partials(cos_pred, mask_gt):
    mesh = plsc.VectorSubcoreMesh(
        core_axis_name="c", subcore_axis_name="s",
        num_cores=_NC, num_subcores=_NS)
    run = pl.kernel(
        _sc_body,
        out_type=jax.ShapeDtypeStruct((_NW, _L), jnp.float32),
        mesh=mesh,
        scratch_types=[
            pltpu.VMEM((2, _CHUNK, _N), jnp.float32),
            pltpu.VMEM((2, _CHUNK, _N), jnp.int32),
            pltpu.VMEM((_L,), jnp.float32),
            pltpu.SemaphoreType.DMA,
            pltpu.SemaphoreType.DMA,
        ],
        compiler_params=pltpu.CompilerParams(needs_layout_passes=False),
    )
    return run(cos_pred, mask_gt)


def _tc_body(cos_ref, mask_ref, out_ref, acc_ref):
    i = pl.program_id(0)

    @pl.when(i == 0)
    def _init():
        acc_ref[0] = 0.0
        acc_ref[1] = 0.0

    c = cos_ref[...]
    m = mask_ref[...].astype(jnp.float32)
    cnt_t = jnp.sum(m, axis=1, keepdims=True)
    cnt_nt = _N - cnt_t
    lt_num = jnp.sum((1.0 - c) * m, axis=1, keepdims=True)
    r = jnp.maximum(c - _MARGIN, 0.0)
    lnt_num = jnp.sum(r, axis=1, keepdims=True) - jnp.sum(
        r * m, axis=1, keepdims=True)
    acc_ref[0] += jnp.sum(lt_num / cnt_t)
    acc_ref[1] += jnp.sum(lnt_num / cnt_nt)

    @pl.when(i == _NBLK_TC - 1)
    def _emit():
        out_ref[0] = acc_ref[0]
        out_ref[1] = acc_ref[1]


def _tc_partials(cos_pred, mask_gt):
    return pl.pallas_call(
        _tc_body,
        grid=(_NBLK_TC,),
        in_specs=[
            pl.BlockSpec((_BM, _N), lambda i: (i, 0)),
            pl.BlockSpec((_BM, _N), lambda i: (i, 0)),
        ],
        out_specs=pl.BlockSpec(memory_space=pltpu.SMEM),
        out_shape=jax.ShapeDtypeStruct((2,), jnp.float32),
        scratch_shapes=[pltpu.SMEM((2,), jnp.float32)],
    )(cos_pred, mask_gt)


def kernel(cos_pred, mask_gt):
    sc = _sc_partials(cos_pred, mask_gt)
    tc = _tc_partials(cos_pred, mask_gt)
    lt_mean = (tc[0] + jnp.sum(sc[:, 0])) / _N
    lnt_mean = (tc[1] + jnp.sum(sc[:, 1])) / _N
    return ((lt_mean + lnt_mean) * 0.5, lt_mean, lnt_mean)


# speedup vs baseline: 3.0192x; 1.0611x over previous
import jax
import jax.numpy as jnp
from jax import lax
from jax.experimental import pallas as pl
from jax.experimental.pallas import tpu as pltpu
from jax.experimental.pallas import tpu_sc as plsc

_MARGIN = 0.1
_N = 4096

_R_SC = 768
_R_TC = _N - _R_SC

_L = 16
_NC = 2
_NS = 16
_NW = _NC * _NS
_RPW = _R_SC // _NW
_CHUNK = 4
_NCH = _RPW // _CHUNK
_SLICES = _N // _L
_UNROLL = 4

_BM = 256
_NBLK_TC = _R_TC // _BM

assert _R_SC % (_NW * _CHUNK) == 0 and _NCH % 2 == 0
assert _R_TC % _BM == 0


def _sc_body(cos_hbm, mask_hbm, out_hbm, cos_buf, mask_buf, res_buf,
             sem0, sem1):
    cid = lax.axis_index("c")
    sid = lax.axis_index("s")
    wid = sid * _NC + cid
    base = _R_TC + wid * _RPW
    lane = lax.iota(jnp.int32, _L)
    zero = jnp.zeros((_L,), jnp.float32)
    sems = (sem0, sem1)

    def start_chunk(k, b):
        r0 = base + k * _CHUNK
        pltpu.make_async_copy(
            cos_hbm.at[pl.ds(r0, _CHUNK)], cos_buf.at[b], sems[b]).start()
        pltpu.make_async_copy(
            mask_hbm.at[pl.ds(r0, _CHUNK)], mask_buf.at[b], sems[b]).start()

    def wait_chunk(b):
        pltpu.make_async_copy(
            cos_hbm.at[pl.ds(base, _CHUNK)], cos_buf.at[b], sems[b]).wait()
        pltpu.make_async_copy(
            mask_hbm.at[pl.ds(base, _CHUNK)], mask_buf.at[b], sems[b]).wait()

    def compute_chunk(b, acc):
        for r in range(_CHUNK):
            def islice(j, c4):
                s_m, s_cm, s_r, s_rm = c4
                for u in range(_UNROLL):
                    sl = pl.ds((j * _UNROLL + u) * _L, _L)
                    c = cos_buf[b, r, sl]
                    mf = mask_buf[b, r, sl].astype(jnp.float32)
                    s_m = s_m + mf
                    s_cm = s_cm + c * mf
                    rr = jnp.maximum(c - _MARGIN, 0.0)
                    s_r = s_r + rr
                    s_rm = s_rm + rr * mf
                return (s_m, s_cm, s_r, s_rm)

            s_m, s_cm, s_r, s_rm = lax.fori_loop(
                0, _SLICES // _UNROLL, islice, (zero, zero, zero, zero))
            cnt = jnp.sum(s_m)
            scm = jnp.sum(s_cm)
            sr = jnp.sum(s_r)
            srm = jnp.sum(s_rm)
            is0 = lane == 0
            num = jnp.where(is0, cnt - scm, sr - srm)
            den = jnp.where(is0, cnt, jnp.float32(_N) - cnt)
            acc = acc + jnp.where(lane < 2, num / den, 0.0)
        return acc

    start_chunk(0, 0)
    start_chunk(1, 1)

    def pair_body(j, acc):
        for b in (0, 1):
            k = j * 2 + b
            wait_chunk(b)
            acc = compute_chunk(b, acc)

            @pl.when(k + 2 < _NCH)
            def _():
                start_chunk(k + 2, b)
        return acc

    acc = lax.fori_loop(0, _NCH // 2, pair_body, zero)
    res_buf[...] = acc
    pltpu.sync_copy(res_buf, out_hbm.at[wid])


def _sc_partials(cos_pred, mask_gt):
    mesh = plsc.VectorSubcoreMesh(
        core_axis_name="c", subcore_axis_name="s",
        num_cores=_NC, num_subcores=_NS)
    run = pl.kernel(
        _sc_body,
        out_type=jax.ShapeDtypeStruct((_NW, _L), jnp.float32),
        mesh=mesh,
        scratch_types=[
            pltpu.VMEM((2, _CHUNK, _N), jnp.float32),
            pltpu.VMEM((2, _CHUNK, _N), jnp.int32),
            pltpu.VMEM((_L,), jnp.float32),
            pltpu.SemaphoreType.DMA,
            pltpu.SemaphoreType.DMA,
        ],
        compiler_params=pltpu.CompilerParams(
            needs_layout_passes=False, skip_device_barrier=True),
    )
    return run(cos_pred, mask_gt)


def _tc_body(cos_ref, mask_ref, out_ref, acc_ref):
    i = pl.program_id(0)

    @pl.when(i == 0)
    def _init():
        acc_ref[0] = 0.0
        acc_ref[1] = 0.0

    c = cos_ref[...]
    m = mask_ref[...].astype(jnp.float32)
    cnt_t = jnp.sum(m, axis=1, keepdims=True)
    cnt_nt = _N - cnt_t
    lt_num = jnp.sum((1.0 - c) * m, axis=1, keepdims=True)
    r = jnp.maximum(c - _MARGIN, 0.0)
    lnt_num = jnp.sum(r, axis=1, keepdims=True) - jnp.sum(
        r * m, axis=1, keepdims=True)
    acc_ref[0] += jnp.sum(lt_num / cnt_t)
    acc_ref[1] += jnp.sum(lnt_num / cnt_nt)

    @pl.when(i == _NBLK_TC - 1)
    def _emit():
        out_ref[0] = acc_ref[0]
        out_ref[1] = acc_ref[1]


def _tc_partials(cos_pred, mask_gt):
    return pl.pallas_call(
        _tc_body,
        grid=(_NBLK_TC,),
        in_specs=[
            pl.BlockSpec((_BM, _N), lambda i: (i, 0)),
            pl.BlockSpec((_BM, _N), lambda i: (i, 0)),
        ],
        out_specs=pl.BlockSpec(memory_space=pltpu.SMEM),
        out_shape=jax.ShapeDtypeStruct((2,), jnp.float32),
        scratch_shapes=[pltpu.SMEM((2,), jnp.float32)],
    )(cos_pred, mask_gt)


def kernel(cos_pred, mask_gt):
    sc = _sc_partials(cos_pred, mask_gt)
    tc = _tc_partials(cos_pred, mask_gt)
    lt_mean = (tc[0] + jnp.sum(sc[:, 0])) / _N
    lnt_mean = (tc[1] + jnp.sum(sc[:, 1])) / _N
    return ((lt_mean + lnt_mean) * 0.5, lt_mean, lnt_mean)


# speedup vs baseline: 4.4702x; 1.4806x over previous
import jax
import jax.numpy as jnp
from jax.experimental import pallas as pl
from jax.experimental.pallas import tpu as pltpu

_MARGIN = 0.1
_N = 4096
_BM = 512
_NBLK = _N // _BM


def _loss_body(cos_ref, mask_ref, out_ref, acc_ref):
    i = pl.program_id(0)

    @pl.when(i == 0)
    def _init():
        acc_ref[0] = 0.0
        acc_ref[1] = 0.0

    c = cos_ref[...]
    m = mask_ref[...].astype(jnp.float32)
    cm = c * m
    r = jnp.maximum(c - _MARGIN, 0.0)
    rm = r * m
    cnt = jnp.sum(m, axis=1, keepdims=True)
    scm = jnp.sum(cm, axis=1, keepdims=True)
    sr = jnp.sum(r, axis=1, keepdims=True)
    srm = jnp.sum(rm, axis=1, keepdims=True)
    lt = (cnt - scm) / cnt
    lnt = (sr - srm) / (_N - cnt)
    acc_ref[0] += jnp.sum(lt)
    acc_ref[1] += jnp.sum(lnt)

    @pl.when(i == _NBLK - 1)
    def _emit():
        lt_mean = acc_ref[0] / _N
        lnt_mean = acc_ref[1] / _N
        out_ref[0] = (lt_mean + lnt_mean) * 0.5
        out_ref[1] = lt_mean
        out_ref[2] = lnt_mean


def kernel(cos_pred, mask_gt):
    out = pl.pallas_call(
        _loss_body,
        grid=(_NBLK,),
        in_specs=[
            pl.BlockSpec((_BM, _N), lambda i: (i, 0)),
            pl.BlockSpec((_BM, _N), lambda i: (i, 0)),
        ],
        out_specs=pl.BlockSpec(memory_space=pltpu.SMEM),
        out_shape=jax.ShapeDtypeStruct((3,), jnp.float32),
        scratch_shapes=[pltpu.SMEM((2,), jnp.float32)],
    )(cos_pred, mask_gt)
    return (out[0], out[1], out[2])
